# baseline (speedup 1.0000x reference)
import jax
import jax.numpy as jnp
from jax.experimental import pallas as pl
from jax.experimental.pallas import tpu as pltpu
from jax.experimental.pallas import tpu_sc as plsc
from jax import lax

NUM_EMBEDDINGS = 1000000
D = 64
HD = D // 2
BATCH = 16384
SEQ = 50
B_TOTAL = BATCH * SEQ

NC = 2
NS = 16
NW = NC * NS

B_PER_W = B_TOTAL // NW
CHUNK = 800
BPC = CHUNK // SEQ
N_CHUNKS = B_PER_W // CHUNK


def _gather_body(ids_hbm, tabl_hbm, tabr_hbm, out_hbm,
                 idx0, idx1, idx2, idx3,
                 rl0, rl1, rr0, rr1,
                 si0, si1, si2, si3,
                 sgl0, sgl1, sgr0, sgr1,
                 sol0, sol1, sor0, sor1):
    idx = [idx0, idx1, idx2, idx3]
    sem_i = [si0, si1, si2, si3]
    tab = [tabl_hbm, tabr_hbm]
    rows = [[rl0, rl1], [rr0, rr1]]
    sem_g = [[sgl0, sgl1], [sgr0, sgr1]]
    sem_o = [[sol0, sol1], [sor0, sor1]]

    wid = lax.axis_index("s") * NC + lax.axis_index("c")
    base = wid * B_PER_W
    base_b = wid * (B_PER_W // SEQ)

    def ids_at(g):
        return ids_hbm.at[pl.ds(base + g * CHUNK, CHUNK)]

    def write_out(g, rb):
        b0 = base_b + g * BPC
        for p in range(2):
            for i in range(BPC):
                pltpu.async_copy(
                    rows[p][rb].at[pl.ds(i * SEQ, SEQ)],
                    out_hbm.at[b0 + i, pl.ds(0, SEQ), pl.ds(p * HD, HD)],
                    sem_o[p][rb])

    def drain_out(rb):
        for p in range(2):
            for _ in range(BPC):
                pltpu.make_async_copy(
                    rows[p][rb].at[pl.ds(0, SEQ)],
                    out_hbm.at[0, pl.ds(0, SEQ), pl.ds(0, HD)],
                    sem_o[p][rb]).wait()

    def gather(b, rb):
        for p in range(2):
            pltpu.async_copy(tab[p].at[idx[b]], rows[p][rb], sem_g[p][rb])

    def drain_gather(b, rb):
        for p in range(2):
            pltpu.make_async_copy(
                tab[p].at[idx[b]], rows[p][rb], sem_g[p][rb]).wait()

    for b in range(4):
        pltpu.async_copy(ids_at(b), idx[b], sem_i[b])

    @pl.loop(0, N_CHUNKS, step=4)
    def _steps(g0):
        for b in range(4):
            g = g0 + b
            rb = b % 2
            pltpu.make_async_copy(ids_at(g), idx[b], sem_i[b]).wait()

            @pl.when(g >= 2)
            def _():
                drain_out(rb)

            gather(b, rb)

            @pl.when(g >= 1)
            def _():
                pb = (b + 1) % 2
                ib = (b + 3) % 4
                drain_gather(ib, pb)
                write_out(g - 1, pb)

                @pl.when(g + 3 < N_CHUNKS)
                def _():
                    pltpu.async_copy(ids_at(g + 3), idx[ib], sem_i[ib])

    last = N_CHUNKS - 1
    drain_gather(3, 1)
    write_out(last, 1)
    drain_out(0)
    drain_out(1)


@jax.jit
def _embedding_gather(ids_flat, tabl, tabr):
    mesh = plsc.VectorSubcoreMesh(
        core_axis_name="c", subcore_axis_name="s", num_cores=NC, num_subcores=NS
    )
    return pl.kernel(
        _gather_body,
        out_type=jax.ShapeDtypeStruct((BATCH, 56, 128), jnp.float32),
        mesh=mesh,
        scratch_types=[
            pltpu.VMEM((CHUNK,), jnp.int32),
            pltpu.VMEM((CHUNK,), jnp.int32),
            pltpu.VMEM((CHUNK,), jnp.int32),
            pltpu.VMEM((CHUNK,), jnp.int32),
            pltpu.VMEM((CHUNK, HD), jnp.float32),
            pltpu.VMEM((CHUNK, HD), jnp.float32),
            pltpu.VMEM((CHUNK, HD), jnp.float32),
            pltpu.VMEM((CHUNK, HD), jnp.float32),
        ] + [pltpu.SemaphoreType.DMA] * 12,
        compiler_params=pltpu.CompilerParams(use_tc_tiling_on_sc=False),
    )(ids_flat, tabl, tabr)


def kernel(input_ids, attention_mask, table):
    ids_flat = input_ids.reshape(-1).astype(jnp.int32)
    out6 = _embedding_gather(ids_flat, table[:, :HD], table[:, HD:])
    return out6[:, :SEQ, :D]


# speedup vs baseline: 1.8486x; 1.8486x over previous
import jax
import jax.numpy as jnp
from jax.experimental import pallas as pl
from jax.experimental.pallas import tpu as pltpu
from jax.experimental.pallas import tpu_sc as plsc
from jax import lax

NUM_EMBEDDINGS = 1000000
D = 64
BATCH = 16384
SEQ = 50
B_TOTAL = BATCH * SEQ

NC = 2
NS = 16
NW = NC * NS

B_PER_W = B_TOTAL // NW
CHUNK = 800
BPC = CHUNK // SEQ
N_CHUNKS = B_PER_W // CHUNK


def _gather_body(ids_hbm, table_hbm, out_hbm,
                 idx0, idx1, idx2, idx3, rows0, rows1,
                 si0, si1, si2, si3, sg0, sg1, so0, so1):
    idx = [idx0, idx1, idx2, idx3]
    sem_i = [si0, si1, si2, si3]
    rows = [rows0, rows1]
    sem_g = [sg0, sg1]
    sem_o = [so0, so1]

    wid = lax.axis_index("s") * NC + lax.axis_index("c")
    base = wid * B_PER_W
    base_b = wid * (B_PER_W // SEQ)

    def ids_at(g):
        return ids_hbm.at[pl.ds(base + g * CHUNK, CHUNK)]

    def write_out(g, rb):
        b0 = base_b + g * BPC
        for i in range(BPC):
            pltpu.async_copy(
                rows[rb].at[pl.ds(i * SEQ, SEQ)],
                out_hbm.at[b0 + i, pl.ds(0, SEQ), pl.ds(0, D)], sem_o[rb])

    def drain_out(rb):
        for _ in range(BPC):
            pltpu.make_async_copy(
                rows[rb].at[pl.ds(0, SEQ)],
                out_hbm.at[0, pl.ds(0, SEQ), pl.ds(0, D)], sem_o[rb]).wait()

    for b in range(4):
        pltpu.async_copy(ids_at(b), idx[b], sem_i[b])

    @pl.loop(0, N_CHUNKS, step=4)
    def _steps(g0):
        for b in range(4):
            g = g0 + b
            rb = b % 2
            pltpu.make_async_copy(ids_at(g), idx[b], sem_i[b]).wait()

            @pl.when(g >= 2)
            def _():
                drain_out(rb)

            pltpu.async_copy(table_hbm.at[idx[b]], rows[rb], sem_g[rb])

            @pl.when(g >= 1)
            def _():
                pb = (b + 1) % 2
                ib = (b + 3) % 4
                pltpu.make_async_copy(
                    table_hbm.at[idx[ib]], rows[pb], sem_g[pb]).wait()
                write_out(g - 1, pb)

                @pl.when(g + 3 < N_CHUNKS)
                def _():
                    pltpu.async_copy(ids_at(g + 3), idx[ib], sem_i[ib])

    last = N_CHUNKS - 1
    pltpu.make_async_copy(table_hbm.at[idx[3]], rows[1], sem_g[1]).wait()
    write_out(last, 1)
    drain_out(0)
    drain_out(1)


@jax.jit
def _embedding_gather(ids_flat, table):
    mesh = plsc.VectorSubcoreMesh(
        core_axis_name="c", subcore_axis_name="s", num_cores=NC, num_subcores=NS
    )
    return pl.kernel(
        _gather_body,
        out_type=jax.ShapeDtypeStruct((BATCH, 56, 128), jnp.float32),
        mesh=mesh,
        scratch_types=[
            pltpu.VMEM((CHUNK,), jnp.int32),
            pltpu.VMEM((CHUNK,), jnp.int32),
            pltpu.VMEM((CHUNK,), jnp.int32),
            pltpu.VMEM((CHUNK,), jnp.int32),
            pltpu.VMEM((CHUNK, D), jnp.float32),
            pltpu.VMEM((CHUNK, D), jnp.float32),
            pltpu.SemaphoreType.DMA,
            pltpu.SemaphoreType.DMA,
            pltpu.SemaphoreType.DMA,
            pltpu.SemaphoreType.DMA,
            pltpu.SemaphoreType.DMA,
            pltpu.SemaphoreType.DMA,
            pltpu.SemaphoreType.DMA,
            pltpu.SemaphoreType.DMA,
        ],
        compiler_params=pltpu.CompilerParams(use_tc_tiling_on_sc=False),
    )(ids_flat, table)


def kernel(input_ids, attention_mask, table):
    ids_flat = input_ids.reshape(-1).astype(jnp.int32)
    out6 = _embedding_gather(ids_flat, table)
    return out6[:, :SEQ, :D]
